# baseline (speedup 1.0000x reference)
import functools

import jax
import jax.numpy as jnp
from jax import lax
from jax.experimental import pallas as pl
from jax.experimental.pallas import tpu as pltpu
from jax.experimental.pallas import tpu_sc as plsc

_N, _E, _F, _H, _C = 10000, 320000, 128, 128, 40
_D3 = 128
_BM = 1000
_R = _N // _BM
_NC, _NS = 2, 16
_NW = _NC * _NS
_EPW = _E // _NW
_K = 80
_NCH = _EPW // _K
_KA = 128
_NCHA = 81
_EPWA = _KA * _NCHA
_EP = _NW * _EPWA
_NP = _N + 16
_ZR = 40
_EPS = 1e-5

_HI = lax.Precision.HIGHEST


def _mesh():
  return plsc.VectorSubcoreMesh(
      core_axis_name="c", subcore_axis_name="s",
      num_cores=_NC, num_subcores=_NS)


def _sc_deg(dst):
  def body(dst_hbm, out_hbm, dall, didx0, didx1, ones, zbuf, acc, semC, semD):
    c = lax.axis_index("c")
    s = lax.axis_index("s")
    wid = s * _NC + c

    @pl.loop(0, 2000 // 16)
    def _z(i):
      zbuf[pl.ds(i * 16, 16)] = jnp.zeros((16,), jnp.float32)

    for j in range(_K // 16):
      ones[pl.ds(j * 16, 16)] = jnp.ones((16,), jnp.float32)

    base = wid * _EPW
    pltpu.sync_copy(dst_hbm.at[pl.ds(base, _EPW)], dall)

    @pl.when(s == 0)
    def _():
      for k in range(_N // 2000):
        pltpu.sync_copy(zbuf, acc.at[pl.ds(k * 2000, 2000)])

    plsc.subcore_barrier()

    def stage(ch, didx):
      for l in range(_K // 16):
        didx[pl.ds(l * 16, 16)] = dall[pl.ds(ch * _K + l * 16, 16)]

    def fire(didx, sem):
      pltpu.async_copy(ones, acc.at[didx], sem, add=True)

    def drain(didx, sem):
      pltpu.make_async_copy(ones, acc.at[didx], sem).wait()

    stage(0, didx0)
    fire(didx0, semC)

    @pl.loop(0, (_NCH - 1) // 2)
    def _edges(j):
      stage(2 * j + 1, didx1)
      fire(didx1, semD)
      drain(didx0, semC)
      stage(2 * j + 2, didx0)
      fire(didx0, semC)
      drain(didx1, semD)

    drain(didx0, semC)
    plsc.subcore_barrier()

    @pl.when(s < _N // 2000)
    def _():
      pltpu.sync_copy(acc.at[pl.ds(s * 2000, 2000)], zbuf)
      pltpu.sync_copy(zbuf, out_hbm.at[pl.ds(c * _N + s * 2000, 2000)])

  return pl.kernel(
      body,
      out_type=jax.ShapeDtypeStruct((_NC * _N,), jnp.float32),
      mesh=_mesh(),
      scratch_types=[
          pltpu.VMEM((_EPW,), jnp.int32),
          pltpu.VMEM((_K,), jnp.int32),
          pltpu.VMEM((_K,), jnp.int32),
          pltpu.VMEM((_K,), jnp.float32),
          pltpu.VMEM((2000,), jnp.float32),
          pltpu.VMEM_SHARED((_N,), jnp.float32),
          pltpu.SemaphoreType.DMA,
          pltpu.SemaphoreType.DMA,
      ],
  )(dst)


def _sc_agg(hs, pk, d):
  def body(hs_hbm, pk_hbm, out_hbm, pall, sidx0, didx0, sidx1, didx1,
           rows0, rows1, zrows, acc, semA, semB):
    c = lax.axis_index("c")
    s = lax.axis_index("s")
    wid = s * _NC + c

    base = wid * _EPWA
    pltpu.async_copy(pk_hbm.at[pl.ds(base, _EPWA)], pall, semB)

    @pl.loop(0, _ZR)
    def _z(i):
      for j in range(d // 16):
        zrows[i, pl.ds(j * 16, 16)] = jnp.zeros((16,), jnp.float32)

    @pl.when(s < _R)
    def _():
      for k in range(_BM // _ZR):
        pltpu.async_copy(zrows, acc.at[pl.ds(s * _BM + k * _ZR, _ZR)], semA)
      for k in range(_BM // _ZR):
        pltpu.make_async_copy(
            zrows, acc.at[pl.ds(s * _BM + k * _ZR, _ZR)], semA).wait()
    pltpu.make_async_copy(pk_hbm.at[pl.ds(base, _EPWA)], pall, semB).wait()
    plsc.subcore_barrier()

    def unpack(ch, sidx, didx):
      for l in range(_KA // 16):
        v = pall[pl.ds(ch * _KA + l * 16, 16)]
        sidx[pl.ds(l * 16, 16)] = v & 0xFFFF
        didx[pl.ds(l * 16, 16)] = lax.shift_right_logical(v, 16)

    def gather(sidx, rows, sem):
      pltpu.async_copy(hs_hbm.at[sidx], rows, sem)

    def wait(rows, sem):
      pltpu.make_async_copy(hs_hbm.at[pl.ds(0, _KA)], rows, sem).wait()

    def scatter(didx, rows):
      pltpu.sync_copy(rows, acc.at[didx], add=True)

    unpack(0, sidx0, didx0)
    gather(sidx0, rows0, semA)
    unpack(1, sidx1, didx1)

    @pl.loop(0, (_NCHA - 1) // 2)
    def _edges(j):
      c0 = 2 * j
      gather(sidx1, rows1, semB)
      wait(rows0, semA)
      scatter(didx0, rows0)
      unpack(c0 + 2, sidx0, didx0)
      gather(sidx0, rows0, semA)
      wait(rows1, semB)
      scatter(didx1, rows1)

      @pl.when(j < (_NCHA - 1) // 2 - 1)
      def _():
        unpack(c0 + 3, sidx1, didx1)

    wait(rows0, semA)
    scatter(didx0, rows0)

    plsc.subcore_barrier()
    @pl.when(s < _R)
    def _():
      wslabs = [(k * _KA, _KA) for k in range(_BM // _KA)]
      wslabs.append(((_BM // _KA) * _KA, _BM - (_BM // _KA) * _KA))
      bufs = [rows0, rows1]
      sems = [semA, semB]
      for k, (r0, nr) in enumerate(wslabs):
        buf, sem = bufs[k % 2], sems[k % 2]
        if k >= 2:
          p0, pn = wslabs[k - 2]
          pltpu.make_async_copy(
              bufs[k % 2].at[pl.ds(0, pn)],
              out_hbm.at[c, pl.ds(s * _BM + p0, pn)], sem).wait()
        pltpu.sync_copy(acc.at[pl.ds(s * _BM + r0, nr)], buf.at[pl.ds(0, nr)])
        pltpu.async_copy(buf.at[pl.ds(0, nr)],
                         out_hbm.at[c, pl.ds(s * _BM + r0, nr)], sem)
      for k in range(len(wslabs) - 2, len(wslabs)):
        r0, nr = wslabs[k]
        pltpu.make_async_copy(bufs[k % 2].at[pl.ds(0, nr)],
                              out_hbm.at[c, pl.ds(s * _BM + r0, nr)],
                              sems[k % 2]).wait()

  return pl.kernel(
      body,
      out_type=jax.ShapeDtypeStruct((_NC, _NP, d), jnp.float32),
      mesh=_mesh(),
      scratch_types=[
          pltpu.VMEM((_EPWA,), jnp.int32),
          pltpu.VMEM((_KA,), jnp.int32),
          pltpu.VMEM((_KA,), jnp.int32),
          pltpu.VMEM((_KA,), jnp.int32),
          pltpu.VMEM((_KA,), jnp.int32),
          pltpu.VMEM((_KA, d), jnp.float32),
          pltpu.VMEM((_KA, d), jnp.float32),
          pltpu.VMEM((_ZR, d), jnp.float32),
          pltpu.VMEM_SHARED((_NP, d), jnp.float32),
          pltpu.SemaphoreType.DMA,
          pltpu.SemaphoreType.DMA,
      ],
  )(hs, pk)


def _tc_mm1(x, W1, degp):
  def body(x_r, w_r, dp_r, h_r, hs_r, di_r):
    deg = dp_r[:, 0] + dp_r[:, 1] + 1.0
    dinv = lax.rsqrt(deg)[:, None]
    h = jnp.dot(x_r[...], w_r[...], preferred_element_type=jnp.float32,
                precision=_HI)
    h_r[...] = h
    hs_r[...] = h * dinv
    di_r[...] = dinv

  return pl.pallas_call(
      body,
      grid=(_R,),
      in_specs=[
          pl.BlockSpec((_BM, _F), lambda i: (i, 0)),
          pl.BlockSpec((_F, _H), lambda i: (0, 0)),
          pl.BlockSpec((_BM, _NC), lambda i: (i, 0)),
      ],
      out_specs=[
          pl.BlockSpec((_BM, _H), lambda i: (i, 0)),
          pl.BlockSpec((_BM, _H), lambda i: (i, 0)),
          pl.BlockSpec((_BM, 1), lambda i: (i, 0)),
      ],
      out_shape=[
          jax.ShapeDtypeStruct((_N, _H), jnp.float32),
          jax.ShapeDtypeStruct((_N, _H), jnp.float32),
          jax.ShapeDtypeStruct((_N, 1), jnp.float32),
      ],
  )(x, W1, degp)


def _tc_layer(ap, h, dinv, b, g, be, W, din, dout):
  def body(ap_r, h_r, di_r, b_r, g_r, be_r, w_r, hout_r, hsout_r,
           t_s, s_s, q_s, di_s):
    p = pl.program_id(0)
    i = pl.program_id(1)

    @pl.when(p == 0)
    def _():
      di = di_r[...]
      t = (ap_r[0] + ap_r[1]) * di + h_r[...] * (di * di) + b_r[...]
      t_s[pl.ds(i * _BM, _BM), :] = t
      di_s[pl.ds(i * _BM, _BM), :] = di
      ps = jnp.sum(t, axis=0, keepdims=True)
      pq = jnp.sum(t * t, axis=0, keepdims=True)

      @pl.when(i == 0)
      def _():
        s_s[...] = ps
        q_s[...] = pq

      @pl.when(i != 0)
      def _():
        s_s[...] += ps
        q_s[...] += pq

    @pl.when(p == 1)
    def _():
      m = s_s[...] / _N
      v = q_s[...] / _N - m * m
      t = t_s[pl.ds(i * _BM, _BM), :]
      xn = (t - m) * lax.rsqrt(v + _EPS) * g_r[...] + be_r[...]
      r = jnp.maximum(xn, 0.0)
      hh = jnp.dot(r, w_r[...], preferred_element_type=jnp.float32,
                   precision=_HI)
      hout_r[...] = hh
      hsout_r[...] = hh * di_s[pl.ds(i * _BM, _BM), :]

  z = lambda p, i: (0, 0)
  p0 = lambda p, i: (jnp.where(p == 0, i, 0), 0)
  p0a = lambda p, i: (0, jnp.where(p == 0, i, 0), 0)
  p1 = lambda p, i: (jnp.where(p == 1, i, 0), 0)
  return pl.pallas_call(
      body,
      grid=(2, _R),
      in_specs=[
          pl.BlockSpec((_NC, _BM, din), p0a),
          pl.BlockSpec((_BM, din), p0),
          pl.BlockSpec((_BM, 1), p0),
          pl.BlockSpec((din,), lambda p, i: (0,)),
          pl.BlockSpec((din,), lambda p, i: (0,)),
          pl.BlockSpec((din,), lambda p, i: (0,)),
          pl.BlockSpec((din, dout), z),
      ],
      out_specs=[
          pl.BlockSpec((_BM, dout), p1),
          pl.BlockSpec((_BM, dout), p1),
      ],
      out_shape=[
          jax.ShapeDtypeStruct((_N, dout), jnp.float32),
          jax.ShapeDtypeStruct((_N, dout), jnp.float32),
      ],
      scratch_shapes=[
          pltpu.VMEM((_N, din), jnp.float32),
          pltpu.VMEM((1, din), jnp.float32),
          pltpu.VMEM((1, din), jnp.float32),
          pltpu.VMEM((_N, 1), jnp.float32),
      ],
  )(ap, h, dinv, b, g, be, W)


def _tc_final(ap, h, dinv, b, g, be):
  def body(ap_r, h_r, di_r, b_r, g_r, be_r, o_r):
    di = di_r[...]
    t = (ap_r[0] + ap_r[1]) * di + h_r[...] * (di * di) + b_r[...]
    mask = lax.broadcasted_iota(jnp.int32, (_BM, _D3), 1) < _C
    tm = jnp.where(mask, t, 0.0)
    m = jnp.sum(tm, axis=1, keepdims=True) / _C
    dc = jnp.where(mask, t - m, 0.0)
    v = jnp.sum(dc * dc, axis=1, keepdims=True) / _C
    y = dc * lax.rsqrt(v + _EPS) * g_r[...] + be_r[...]
    ymax = jnp.max(jnp.where(mask, y, -jnp.inf), axis=1, keepdims=True)
    e = jnp.where(mask, jnp.exp(y - ymax), 0.0)
    lse = jnp.log(jnp.sum(e, axis=1, keepdims=True))
    z = y - ymax - lse
    o_r[...] = z[:, :_C]

  return pl.pallas_call(
      body,
      grid=(_R,),
      in_specs=[
          pl.BlockSpec((_NC, _BM, _D3), lambda i: (0, i, 0)),
          pl.BlockSpec((_BM, _D3), lambda i: (i, 0)),
          pl.BlockSpec((_BM, 1), lambda i: (i, 0)),
          pl.BlockSpec((_D3,), lambda i: (0,)),
          pl.BlockSpec((_D3,), lambda i: (0,)),
          pl.BlockSpec((_D3,), lambda i: (0,)),
      ],
      out_specs=pl.BlockSpec((_BM, _C), lambda i: (i, 0)),
      out_shape=jax.ShapeDtypeStruct((_N, _C), jnp.float32),
  )(ap, h, dinv, b, g, be)


def kernel(x, edge_index, W1, b1, W2, b2, W3, b3, g1, be1, g2, be2, g3, be3):
  src = edge_index[0]
  dst = edge_index[1]
  W3p = jnp.pad(W3, ((0, 0), (0, _D3 - _C)))
  b3p = jnp.pad(b3, (0, _D3 - _C))
  g3p = jnp.pad(g3, (0, _D3 - _C))
  be3p = jnp.pad(be3, (0, _D3 - _C))

  pk = src | (dst << 16)
  padvals = (_N + (jnp.arange(_EP - _E, dtype=jnp.int32) % 16)) << 16
  pkp = jnp.concatenate([pk, padvals])

  degp = _sc_deg(dst).reshape(_NC, _N).T
  h1, hs1, dinv = _tc_mm1(x, W1, degp)
  a1 = _sc_agg(hs1, pkp, _H)
  h2, hs2 = _tc_layer(a1, h1, dinv, b1, g1, be1, W2, _H, _H)
  a2 = _sc_agg(hs2, pkp, _H)
  h3, hs3 = _tc_layer(a2, h2, dinv, b2, g2, be2, W3p, _H, _D3)
  a3 = _sc_agg(hs3, pkp, _D3)
  return _tc_final(a3, h3, dinv, b3p, g3p, be3p)


# speedup vs baseline: 3.9130x; 3.9130x over previous
import functools

import jax
import jax.numpy as jnp
from jax import lax
from jax.experimental import pallas as pl
from jax.experimental.pallas import tpu as pltpu
from jax.experimental.pallas import tpu_sc as plsc

_N, _E, _F, _H, _C = 10000, 320000, 128, 128, 40
_D3 = 128
_BM = 1000
_R = _N // _BM
_NC, _NS = 2, 16
_NW = _NC * _NS
_EPW = _E // _NW
_K = 80
_NCH = _EPW // _K
_ZR = 120
_EPS = 1e-5

_HI = lax.Precision.HIGHEST


def _mesh():
  return plsc.VectorSubcoreMesh(
      core_axis_name="c", subcore_axis_name="s",
      num_cores=_NC, num_subcores=_NS)


def _sc_deg(dst):
  def body(dst_hbm, out_hbm, dall, didx0, didx1, ones, zbuf, acc, semC, semD):
    c = lax.axis_index("c")
    s = lax.axis_index("s")
    wid = s * _NC + c

    @pl.loop(0, 2000 // 16)
    def _z(i):
      zbuf[pl.ds(i * 16, 16)] = jnp.zeros((16,), jnp.float32)

    for j in range(_K // 16):
      ones[pl.ds(j * 16, 16)] = jnp.ones((16,), jnp.float32)

    base = wid * _EPW
    pltpu.sync_copy(dst_hbm.at[pl.ds(base, _EPW)], dall)

    @pl.when(s == 0)
    def _():
      for k in range(_N // 2000):
        pltpu.sync_copy(zbuf, acc.at[pl.ds(k * 2000, 2000)])

    plsc.subcore_barrier()

    def stage(ch, didx):
      for l in range(_K // 16):
        didx[pl.ds(l * 16, 16)] = dall[pl.ds(ch * _K + l * 16, 16)]

    def fire(didx, sem):
      pltpu.async_copy(ones, acc.at[didx], sem, add=True)

    def drain(didx, sem):
      pltpu.make_async_copy(ones, acc.at[didx], sem).wait()

    stage(0, didx0)
    fire(didx0, semC)

    @pl.loop(0, (_NCH - 1) // 2)
    def _edges(j):
      stage(2 * j + 1, didx1)
      fire(didx1, semD)
      drain(didx0, semC)
      stage(2 * j + 2, didx0)
      fire(didx0, semC)
      drain(didx1, semD)

    drain(didx0, semC)
    plsc.subcore_barrier()

    @pl.when(s < _N // 2000)
    def _():
      pltpu.sync_copy(acc.at[pl.ds(s * 2000, 2000)], zbuf)
      pltpu.sync_copy(zbuf, out_hbm.at[pl.ds(c * _N + s * 2000, 2000)])

  return pl.kernel(
      body,
      out_type=jax.ShapeDtypeStruct((_NC * _N,), jnp.float32),
      mesh=_mesh(),
      scratch_types=[
          pltpu.VMEM((_EPW,), jnp.int32),
          pltpu.VMEM((_K,), jnp.int32),
          pltpu.VMEM((_K,), jnp.int32),
          pltpu.VMEM((_K,), jnp.float32),
          pltpu.VMEM((2000,), jnp.float32),
          pltpu.VMEM_SHARED((_N,), jnp.float32),
          pltpu.SemaphoreType.DMA,
          pltpu.SemaphoreType.DMA,
      ],
  )(dst)


def _sc_agg(hs, pk, d):
  def body(hs_hbm, pk_hbm, out_hbm, pall, sidx0, didx0, sidx1, didx1,
           rows0, rows1, zrows, acc, semA, semB, semC):
    c = lax.axis_index("c")
    s = lax.axis_index("s")
    wid = s * _NC + c

    base = wid * _EPW
    pltpu.async_copy(pk_hbm.at[pl.ds(base, _EPW)], pall, semB)

    @pl.loop(0, _ZR)
    def _z(i):
      for j in range(d // 16):
        zrows[i, pl.ds(j * 16, 16)] = jnp.zeros((16,), jnp.float32)

    def slabs(fn):
      for k in range(_BM // _ZR):
        fn(k * _ZR, _ZR)
      fn((_BM // _ZR) * _ZR, _BM - (_BM // _ZR) * _ZR)

    @pl.when(s < _R)
    def _():
      slabs(lambda r0, nr: pltpu.async_copy(
          zrows.at[pl.ds(0, nr)], acc.at[pl.ds(s * _BM + r0, nr)], semA))
      slabs(lambda r0, nr: pltpu.make_async_copy(
          zrows.at[pl.ds(0, nr)], acc.at[pl.ds(s * _BM + r0, nr)], semA).wait())
    pltpu.make_async_copy(pk_hbm.at[pl.ds(base, _EPW)], pall, semB).wait()
    plsc.subcore_barrier()

    def unpack(ch, sidx, didx):
      for l in range(_K // 16):
        v = pall[pl.ds(ch * _K + l * 16, 16)]
        sidx[pl.ds(l * 16, 16)] = v & 0xFFFF
        didx[pl.ds(l * 16, 16)] = lax.shift_right_logical(v, 16)

    def gather(sidx, rows, sem):
      pltpu.async_copy(hs_hbm.at[sidx], rows, sem)

    def wait(rows, sem):
      pltpu.make_async_copy(hs_hbm.at[pl.ds(0, _K)], rows, sem).wait()

    def scatter(didx, rows):
      pltpu.sync_copy(rows, acc.at[didx], add=True)

    unpack(0, sidx0, didx0)
    gather(sidx0, rows0, semA)
    unpack(1, sidx1, didx1)

    @pl.loop(0, (_NCH - 1) // 2)
    def _edges(j):
      c0 = 2 * j
      gather(sidx1, rows1, semB)
      wait(rows0, semA)
      scatter(didx0, rows0)
      unpack(c0 + 2, sidx0, didx0)
      gather(sidx0, rows0, semA)
      wait(rows1, semB)
      scatter(didx1, rows1)

      @pl.when(j < (_NCH - 1) // 2 - 1)
      def _():
        unpack(c0 + 3, sidx1, didx1)

    wait(rows0, semA)
    scatter(didx0, rows0)

    plsc.subcore_barrier()
    @pl.when(s < _R)
    def _():
      wslabs = [(k * _K, _K) for k in range(_BM // _K)]
      wslabs.append(((_BM // _K) * _K, _BM - (_BM // _K) * _K))
      bufs = [rows0, rows1, zrows]
      sems = [semA, semB, semC]
      for k, (r0, nr) in enumerate(wslabs):
        buf, sem = bufs[k % 3], sems[k % 3]
        if k >= 3:
          p0, pn = wslabs[k - 3]
          pltpu.make_async_copy(
              bufs[(k - 3) % 3].at[pl.ds(0, pn)],
              out_hbm.at[c, pl.ds(s * _BM + p0, pn)], sems[(k - 3) % 3]).wait()
        pltpu.sync_copy(acc.at[pl.ds(s * _BM + r0, nr)], buf.at[pl.ds(0, nr)])
        pltpu.async_copy(buf.at[pl.ds(0, nr)],
                         out_hbm.at[c, pl.ds(s * _BM + r0, nr)], sem)
      for k in range(len(wslabs) - 3, len(wslabs)):
        r0, nr = wslabs[k]
        pltpu.make_async_copy(bufs[k % 3].at[pl.ds(0, nr)],
                              out_hbm.at[c, pl.ds(s * _BM + r0, nr)],
                              sems[k % 3]).wait()

  return pl.kernel(
      body,
      out_type=jax.ShapeDtypeStruct((_NC, _N, d), jnp.float32),
      mesh=_mesh(),
      scratch_types=[
          pltpu.VMEM((_EPW,), jnp.int32),
          pltpu.VMEM((_K,), jnp.int32),
          pltpu.VMEM((_K,), jnp.int32),
          pltpu.VMEM((_K,), jnp.int32),
          pltpu.VMEM((_K,), jnp.int32),
          pltpu.VMEM((_K, d), jnp.float32),
          pltpu.VMEM((_K, d), jnp.float32),
          pltpu.VMEM((_ZR, d), jnp.float32),
          pltpu.VMEM_SHARED((_N, d), jnp.float32),
          pltpu.SemaphoreType.DMA,
          pltpu.SemaphoreType.DMA,
          pltpu.SemaphoreType.DMA,
      ],
  )(hs, pk)


def _tc_mm1(x, W1, degp):
  def body(x_r, w_r, dp_r, h_r, hs_r, di_r):
    deg = dp_r[:, 0] + dp_r[:, 1] + 1.0
    dinv = lax.rsqrt(deg)[:, None]
    h = jnp.dot(x_r[...], w_r[...], preferred_element_type=jnp.float32,
                precision=_HI)
    h_r[...] = h
    hs_r[...] = h * dinv
    di_r[...] = dinv

  return pl.pallas_call(
      body,
      grid=(_R,),
      in_specs=[
          pl.BlockSpec((_BM, _F), lambda i: (i, 0)),
          pl.BlockSpec((_F, _H), lambda i: (0, 0)),
          pl.BlockSpec((_BM, _NC), lambda i: (i, 0)),
      ],
      out_specs=[
          pl.BlockSpec((_BM, _H), lambda i: (i, 0)),
          pl.BlockSpec((_BM, _H), lambda i: (i, 0)),
          pl.BlockSpec((_BM, 1), lambda i: (i, 0)),
      ],
      out_shape=[
          jax.ShapeDtypeStruct((_N, _H), jnp.float32),
          jax.ShapeDtypeStruct((_N, _H), jnp.float32),
          jax.ShapeDtypeStruct((_N, 1), jnp.float32),
      ],
  )(x, W1, degp)


def _tc_layer(ap, h, dinv, b, g, be, W, din, dout):
  def body(ap_r, h_r, di_r, b_r, g_r, be_r, w_r, hout_r, hsout_r,
           t_s, s_s, q_s, di_s):
    p = pl.program_id(0)
    i = pl.program_id(1)

    @pl.when(p == 0)
    def _():
      di = di_r[...]
      t = (ap_r[0] + ap_r[1]) * di + h_r[...] * (di * di) + b_r[...]
      t_s[pl.ds(i * _BM, _BM), :] = t
      di_s[pl.ds(i * _BM, _BM), :] = di
      ps = jnp.sum(t, axis=0, keepdims=True)
      pq = jnp.sum(t * t, axis=0, keepdims=True)

      @pl.when(i == 0)
      def _():
        s_s[...] = ps
        q_s[...] = pq

      @pl.when(i != 0)
      def _():
        s_s[...] += ps
        q_s[...] += pq

    @pl.when(p == 1)
    def _():
      m = s_s[...] / _N
      v = q_s[...] / _N - m * m
      t = t_s[pl.ds(i * _BM, _BM), :]
      xn = (t - m) * lax.rsqrt(v + _EPS) * g_r[...] + be_r[...]
      r = jnp.maximum(xn, 0.0)
      hh = jnp.dot(r, w_r[...], preferred_element_type=jnp.float32,
                   precision=_HI)
      hout_r[...] = hh
      hsout_r[...] = hh * di_s[pl.ds(i * _BM, _BM), :]

  z = lambda p, i: (0, 0)
  p0 = lambda p, i: (jnp.where(p == 0, i, 0), 0)
  p0a = lambda p, i: (0, jnp.where(p == 0, i, 0), 0)
  p1 = lambda p, i: (jnp.where(p == 1, i, 0), 0)
  return pl.pallas_call(
      body,
      grid=(2, _R),
      in_specs=[
          pl.BlockSpec((_NC, _BM, din), p0a),
          pl.BlockSpec((_BM, din), p0),
          pl.BlockSpec((_BM, 1), p0),
          pl.BlockSpec((din,), lambda p, i: (0,)),
          pl.BlockSpec((din,), lambda p, i: (0,)),
          pl.BlockSpec((din,), lambda p, i: (0,)),
          pl.BlockSpec((din, dout), z),
      ],
      out_specs=[
          pl.BlockSpec((_BM, dout), p1),
          pl.BlockSpec((_BM, dout), p1),
      ],
      out_shape=[
          jax.ShapeDtypeStruct((_N, dout), jnp.float32),
          jax.ShapeDtypeStruct((_N, dout), jnp.float32),
      ],
      scratch_shapes=[
          pltpu.VMEM((_N, din), jnp.float32),
          pltpu.VMEM((1, din), jnp.float32),
          pltpu.VMEM((1, din), jnp.float32),
          pltpu.VMEM((_N, 1), jnp.float32),
      ],
  )(ap, h, dinv, b, g, be, W)


def _tc_final(ap, h, dinv, b, g, be):
  def body(ap_r, h_r, di_r, b_r, g_r, be_r, o_r):
    di = di_r[...]
    t = (ap_r[0] + ap_r[1]) * di + h_r[...] * (di * di) + b_r[...]
    mask = lax.broadcasted_iota(jnp.int32, (_BM, _D3), 1) < _C
    tm = jnp.where(mask, t, 0.0)
    m = jnp.sum(tm, axis=1, keepdims=True) / _C
    dc = jnp.where(mask, t - m, 0.0)
    v = jnp.sum(dc * dc, axis=1, keepdims=True) / _C
    y = dc * lax.rsqrt(v + _EPS) * g_r[...] + be_r[...]
    ymax = jnp.max(jnp.where(mask, y, -jnp.inf), axis=1, keepdims=True)
    e = jnp.where(mask, jnp.exp(y - ymax), 0.0)
    lse = jnp.log(jnp.sum(e, axis=1, keepdims=True))
    z = y - ymax - lse
    o_r[...] = z[:, :_C]

  return pl.pallas_call(
      body,
      grid=(_R,),
      in_specs=[
          pl.BlockSpec((_NC, _BM, _D3), lambda i: (0, i, 0)),
          pl.BlockSpec((_BM, _D3), lambda i: (i, 0)),
          pl.BlockSpec((_BM, 1), lambda i: (i, 0)),
          pl.BlockSpec((_D3,), lambda i: (0,)),
          pl.BlockSpec((_D3,), lambda i: (0,)),
          pl.BlockSpec((_D3,), lambda i: (0,)),
      ],
      out_specs=pl.BlockSpec((_BM, _C), lambda i: (i, 0)),
      out_shape=jax.ShapeDtypeStruct((_N, _C), jnp.float32),
  )(ap, h, dinv, b, g, be)


def kernel(x, edge_index, W1, b1, W2, b2, W3, b3, g1, be1, g2, be2, g3, be3):
  src = edge_index[0]
  dst = edge_index[1]
  W3p = jnp.pad(W3, ((0, 0), (0, _D3 - _C)))
  b3p = jnp.pad(b3, (0, _D3 - _C))
  g3p = jnp.pad(g3, (0, _D3 - _C))
  be3p = jnp.pad(be3, (0, _D3 - _C))

  pk = src | (dst << 16)

  degp = _sc_deg(dst).reshape(_NC, _N).T
  h1, hs1, dinv = _tc_mm1(x, W1, degp)
  a1 = _sc_agg(hs1, pk, _H)
  h2, hs2 = _tc_layer(a1, h1, dinv, b1, g1, be1, W2, _H, _H)
  a2 = _sc_agg(hs2, pk, _H)
  h3, hs3 = _tc_layer(a2, h2, dinv, b2, g2, be2, W3p, _H, _D3)
  a3 = _sc_agg(hs3, pk, _D3)
  return _tc_final(a3, h3, dinv, b3p, g3p, be3p)
